# baseline (speedup 1.0000x reference)
import jax
import jax.numpy as jnp
from jax import lax
from jax.experimental import pallas as pl
from jax.experimental.pallas import tpu as pltpu
from jax.experimental.pallas import tpu_sc as plsc

N_FACTORS = 50
BATCH = 16384
NC, NS, L = 2, 16, 16
NW = NC * NS
BPW = BATCH // NW
NG = BPW // L
C = 256
NGH = C // L


def _body(iu_hbm, im_hbm, u_hbm, m_hbm, o_hbm,
          iu_v, im_v, u_rows, m_rows, out_v, sem):
    wid = lax.axis_index("s") * NC + lax.axis_index("c")
    base = wid * BPW

    pltpu.sync_copy(iu_hbm.at[pl.ds(base, BPW)], iu_v)
    pltpu.sync_copy(im_hbm.at[pl.ds(base, BPW)], im_v)

    lanes = lax.iota(jnp.int32, L)

    def half(h, _):
        hb = h * C

        def issue(g, _):
            vu = iu_v[pl.ds(hb + g * L, L)]
            vm = im_v[pl.ds(hb + g * L, L)]
            for j in range(L):
                i = g * L + j
                pltpu.async_copy(u_hbm.at[pl.ds(vu[j], 1), :],
                                 u_rows.at[pl.ds(i, 1), :], sem)
                pltpu.async_copy(m_hbm.at[pl.ds(vm[j], 1), :],
                                 m_rows.at[pl.ds(i, 1), :], sem)
            return 0

        lax.fori_loop(0, NGH, issue, 0)

        pltpu.make_async_copy(u_hbm.at[pl.ds(0, C), :], u_rows, sem).wait()
        pltpu.make_async_copy(m_hbm.at[pl.ds(0, C), :], m_rows, sem).wait()

        def dot(g, _):
            rows = g * L + lanes
            acc = (plsc.load_gather(u_rows, [rows, jnp.zeros((L,), jnp.int32)])
                   * plsc.load_gather(m_rows, [rows, jnp.zeros((L,), jnp.int32)]))
            for f in range(1, N_FACTORS):
                cols = jnp.full((L,), f, jnp.int32)
                acc = acc + (plsc.load_gather(u_rows, [rows, cols])
                             * plsc.load_gather(m_rows, [rows, cols]))
            out_v[pl.ds(hb + g * L, L)] = acc
            return 0

        lax.fori_loop(0, NGH, dot, 0)
        return 0

    lax.fori_loop(0, BPW // C, half, 0)

    pltpu.sync_copy(out_v, o_hbm.at[pl.ds(base, BPW)])


@jax.jit
def _embedding_dot(cats, u_weight, m_weight):
    users = cats[:, 0]
    movies = cats[:, 1]
    mesh = plsc.VectorSubcoreMesh(core_axis_name="c", subcore_axis_name="s")
    run = pl.kernel(
        _body, mesh=mesh,
        compiler_params=pltpu.CompilerParams(needs_layout_passes=False),
        out_type=jax.ShapeDtypeStruct((BATCH,), jnp.float32),
        scratch_types=[
            pltpu.VMEM((BPW,), jnp.int32),
            pltpu.VMEM((BPW,), jnp.int32),
            pltpu.VMEM((C, N_FACTORS), jnp.float32),
            pltpu.VMEM((C, N_FACTORS), jnp.float32),
            pltpu.VMEM((BPW,), jnp.float32),
            pltpu.SemaphoreType.DMA,
        ],
    )
    return run(users, movies, u_weight, m_weight)


def kernel(cats, conts, u_weight, m_weight):
    del conts
    return _embedding_dot(cats.astype(jnp.int32), u_weight, m_weight)
